# baseline (device time: 30327 ns/iter reference)
import jax
import jax.numpy as jnp
from jax import lax
from jax.experimental import pallas as pl
from jax.experimental.pallas import tpu as pltpu

E_LOC = 2


def kernel(x, router, W1, W2):
    t_loc, d = x.shape

    def body(x_ref, r_ref, w1_ref, w2_ref, out_ref,
             xr_buf, rr_buf, csend_buf, crecv_buf, send_sems, recv_sems):
        my_x = lax.axis_index("x")
        my_y = lax.axis_index("y")
        nbr = (1 - my_x, my_y)

        barrier_sem = pltpu.get_barrier_semaphore()
        pl.semaphore_signal(barrier_sem, inc=1, device_id=nbr,
                            device_id_type=pl.DeviceIdType.MESH)
        pl.semaphore_wait(barrier_sem, 1)

        rdma_x = pltpu.make_async_remote_copy(
            src_ref=x_ref, dst_ref=xr_buf,
            send_sem=send_sems.at[0], recv_sem=recv_sems.at[0],
            device_id=nbr, device_id_type=pl.DeviceIdType.MESH)
        rdma_x.start()
        rdma_r = pltpu.make_async_remote_copy(
            src_ref=r_ref, dst_ref=rr_buf,
            send_sem=send_sems.at[1], recv_sem=recv_sems.at[1],
            device_id=nbr, device_id_type=pl.DeviceIdType.MESH)
        rdma_r.start()
        rdma_x.wait()
        rdma_r.wait()

        x_all = jnp.concatenate([x_ref[...], xr_buf[...]], axis=0)

        g_mine = jnp.dot(x_all, r_ref[...], preferred_element_type=jnp.float32)
        g_other = jnp.dot(x_all, rr_buf[...], preferred_element_type=jnp.float32)
        g4 = jnp.concatenate([g_mine, g_other], axis=1)
        m1 = jnp.max(g4, axis=1, keepdims=True)
        m2 = jnp.max(jnp.where(g4 == m1, -1e30, g4), axis=1, keepdims=True)
        denom = 1.0 + jnp.exp(m2 - m1)

        contrib = jnp.zeros((2 * t_loc, d), jnp.float32)
        for e in range(E_LOC):
            g = g_mine[:, e:e + 1]
            w = jnp.where(g >= m2, jnp.exp(g - m1) / denom, 0.0)
            h = jnp.maximum(
                jnp.dot(x_all, w1_ref[e], preferred_element_type=jnp.float32),
                0.0)
            c = jnp.dot(h, w2_ref[e], preferred_element_type=jnp.float32)
            contrib = contrib + c * w

        csend_buf[...] = contrib[t_loc:, :]
        rdma_c = pltpu.make_async_remote_copy(
            src_ref=csend_buf, dst_ref=crecv_buf,
            send_sem=send_sems.at[2], recv_sem=recv_sems.at[2],
            device_id=nbr, device_id_type=pl.DeviceIdType.MESH)
        rdma_c.start()
        rdma_c.wait()

        out_ref[...] = contrib[:t_loc, :] + crecv_buf[...]

    return pl.pallas_call(
        body,
        out_shape=jax.ShapeDtypeStruct((t_loc, d), jnp.float32),
        in_specs=[pl.BlockSpec(memory_space=pltpu.VMEM)] * 4,
        out_specs=pl.BlockSpec(memory_space=pltpu.VMEM),
        scratch_shapes=[
            pltpu.VMEM((t_loc, d), jnp.float32),
            pltpu.VMEM(router.shape, jnp.float32),
            pltpu.VMEM((t_loc, d), jnp.float32),
            pltpu.VMEM((t_loc, d), jnp.float32),
            pltpu.SemaphoreType.DMA((3,)),
            pltpu.SemaphoreType.DMA((3,)),
        ],
        compiler_params=pltpu.CompilerParams(collective_id=0),
    )(x, router, W1, W2)


# device time: 28953 ns/iter; 1.0475x vs baseline; 1.0475x over previous
import jax
import jax.numpy as jnp
from jax import lax
from jax.experimental import pallas as pl
from jax.experimental.pallas import tpu as pltpu

E_LOC = 2


def kernel(x, router, W1, W2):
    t_loc, d = x.shape

    def body(x_ref, r_ref, w1_ref, w2_ref, out_ref,
             xr_buf, rr_buf, csend_buf, crecv_buf, send_sems, recv_sems):
        my_x = lax.axis_index("x")
        my_y = lax.axis_index("y")
        nbr = (1 - my_x, my_y)

        barrier_sem = pltpu.get_barrier_semaphore()
        pl.semaphore_signal(barrier_sem, inc=1, device_id=nbr,
                            device_id_type=pl.DeviceIdType.MESH)
        pl.semaphore_wait(barrier_sem, 1)

        rdma_x = pltpu.make_async_remote_copy(
            src_ref=x_ref, dst_ref=xr_buf,
            send_sem=send_sems.at[0], recv_sem=recv_sems.at[0],
            device_id=nbr, device_id_type=pl.DeviceIdType.MESH)
        rdma_x.start()
        rdma_r = pltpu.make_async_remote_copy(
            src_ref=r_ref, dst_ref=rr_buf,
            send_sem=send_sems.at[1], recv_sem=recv_sems.at[1],
            device_id=nbr, device_id_type=pl.DeviceIdType.MESH)
        rdma_r.start()

        def topk_weights(g_loc, g_rem):
            g4 = jnp.concatenate([g_loc, g_rem], axis=1)
            m1 = jnp.max(g4, axis=1, keepdims=True)
            m2 = jnp.max(jnp.where(g4 == m1, -1e30, g4), axis=1,
                         keepdims=True)
            denom = 1.0 + jnp.exp(m2 - m1)
            return [jnp.where(g_loc[:, e:e + 1] >= m2,
                              jnp.exp(g_loc[:, e:e + 1] - m1) / denom, 0.0)
                    for e in range(E_LOC)]

        w1b = [w1_ref[e].astype(jnp.bfloat16) for e in range(E_LOC)]
        w2b = [w2_ref[e].astype(jnp.bfloat16) for e in range(E_LOC)]

        def expert_out(x_f32):
            xb = x_f32.astype(jnp.bfloat16)
            cs = []
            for e in range(E_LOC):
                h = jnp.maximum(
                    jnp.dot(xb, w1b[e],
                            preferred_element_type=jnp.float32), 0.0)
                cs.append(jnp.dot(h.astype(jnp.bfloat16), w2b[e],
                                  preferred_element_type=jnp.float32))
            return cs

        x_my = x_ref[...]
        c_my = expert_out(x_my)

        rdma_r.wait()
        rdma_x.wait()

        x_nb = xr_buf[...]
        g_nb_loc = jnp.dot(x_nb, r_ref[...],
                           preferred_element_type=jnp.float32)
        g_nb_rem = jnp.dot(x_nb, rr_buf[...],
                           preferred_element_type=jnp.float32)
        w_nb = topk_weights(g_nb_loc, g_nb_rem)
        c_nb = expert_out(x_nb)
        csend_buf[...] = c_nb[0] * w_nb[0] + c_nb[1] * w_nb[1]
        rdma_c = pltpu.make_async_remote_copy(
            src_ref=csend_buf, dst_ref=crecv_buf,
            send_sem=send_sems.at[2], recv_sem=recv_sems.at[2],
            device_id=nbr, device_id_type=pl.DeviceIdType.MESH)
        rdma_c.start()

        g_my_loc = jnp.dot(x_my, r_ref[...],
                           preferred_element_type=jnp.float32)
        g_my_rem = jnp.dot(x_my, rr_buf[...],
                           preferred_element_type=jnp.float32)
        w_my = topk_weights(g_my_loc, g_my_rem)
        out_mine = c_my[0] * w_my[0] + c_my[1] * w_my[1]

        rdma_c.wait()
        out_ref[...] = out_mine + crecv_buf[...]

    return pl.pallas_call(
        body,
        out_shape=jax.ShapeDtypeStruct((t_loc, d), jnp.float32),
        in_specs=[pl.BlockSpec(memory_space=pltpu.VMEM)] * 4,
        out_specs=pl.BlockSpec(memory_space=pltpu.VMEM),
        scratch_shapes=[
            pltpu.VMEM((t_loc, d), jnp.float32),
            pltpu.VMEM(router.shape, jnp.float32),
            pltpu.VMEM((t_loc, d), jnp.float32),
            pltpu.VMEM((t_loc, d), jnp.float32),
            pltpu.SemaphoreType.DMA((3,)),
            pltpu.SemaphoreType.DMA((3,)),
        ],
        compiler_params=pltpu.CompilerParams(collective_id=0),
    )(x, router, W1, W2)


# device time: 23973 ns/iter; 1.2650x vs baseline; 1.2077x over previous
import jax
import jax.numpy as jnp
from jax import lax
from jax.experimental import pallas as pl
from jax.experimental.pallas import tpu as pltpu

E_LOC = 2


def kernel(x, router, W1, W2):
    t_loc, d = x.shape
    f = W1.shape[2]

    def body(x_ref, r_ref, w1_hbm, w2_hbm, out_ref,
             w1_vmem, w2_vmem, rr_buf, xsend, xrecv, csend, crecv,
             w_sems, send_sems, recv_sems):
        my_x = lax.axis_index("x")
        my_y = lax.axis_index("y")
        nbr = (1 - my_x, my_y)

        w_copies = []
        for e in range(E_LOC):
            c1 = pltpu.make_async_copy(w1_hbm.at[e], w1_vmem.at[e],
                                       w_sems.at[e])
            c1.start()
            c2 = pltpu.make_async_copy(w2_hbm.at[e], w2_vmem.at[e],
                                       w_sems.at[E_LOC + e])
            c2.start()
            w_copies.append((c1, c2))

        barrier_sem = pltpu.get_barrier_semaphore()
        pl.semaphore_signal(barrier_sem, inc=1, device_id=nbr,
                            device_id_type=pl.DeviceIdType.MESH)
        pl.semaphore_wait(barrier_sem, 1)

        xsend[...] = x_ref[...].astype(jnp.bfloat16)
        rdma_x = pltpu.make_async_remote_copy(
            src_ref=xsend, dst_ref=xrecv,
            send_sem=send_sems.at[0], recv_sem=recv_sems.at[0],
            device_id=nbr, device_id_type=pl.DeviceIdType.MESH)
        rdma_x.start()
        rdma_r = pltpu.make_async_remote_copy(
            src_ref=r_ref, dst_ref=rr_buf,
            send_sem=send_sems.at[1], recv_sem=recv_sems.at[1],
            device_id=nbr, device_id_type=pl.DeviceIdType.MESH)
        rdma_r.start()

        def topk_weights(g_loc, g_rem):
            g4 = jnp.concatenate([g_loc, g_rem], axis=1)
            m1 = jnp.max(g4, axis=1, keepdims=True)
            m2 = jnp.max(jnp.where(g4 == m1, -1e30, g4), axis=1,
                         keepdims=True)
            denom = 1.0 + jnp.exp(m2 - m1)
            return [jnp.where(g_loc[:, e:e + 1] >= m2,
                              jnp.exp(g_loc[:, e:e + 1] - m1) / denom, 0.0)
                    for e in range(E_LOC)]

        def expert_out(x_f32, wait_weights=False):
            cs = []
            for e in range(E_LOC):
                if wait_weights:
                    w_copies[e][0].wait()
                h = jnp.maximum(
                    jnp.dot(x_f32, w1_vmem[e],
                            preferred_element_type=jnp.float32), 0.0)
                if wait_weights:
                    w_copies[e][1].wait()
                cs.append(jnp.dot(h, w2_vmem[e],
                                  preferred_element_type=jnp.float32))
            return cs

        x_my = x_ref[...]
        c_my = expert_out(x_my, wait_weights=True)

        rdma_r.wait()
        rdma_x.wait()

        x_nb = xrecv[...].astype(jnp.float32)
        g_nb_loc = jnp.dot(x_nb, r_ref[...],
                           preferred_element_type=jnp.float32)
        g_nb_rem = jnp.dot(x_nb, rr_buf[...],
                           preferred_element_type=jnp.float32)
        w_nb = topk_weights(g_nb_loc, g_nb_rem)
        c_nb = expert_out(x_nb)
        csend[...] = (c_nb[0] * w_nb[0] + c_nb[1] * w_nb[1]).astype(
            jnp.bfloat16)
        rdma_c = pltpu.make_async_remote_copy(
            src_ref=csend, dst_ref=crecv,
            send_sem=send_sems.at[2], recv_sem=recv_sems.at[2],
            device_id=nbr, device_id_type=pl.DeviceIdType.MESH)
        rdma_c.start()

        g_my_loc = jnp.dot(x_my, r_ref[...],
                           preferred_element_type=jnp.float32)
        g_my_rem = jnp.dot(x_my, rr_buf[...],
                           preferred_element_type=jnp.float32)
        w_my = topk_weights(g_my_loc, g_my_rem)
        out_mine = c_my[0] * w_my[0] + c_my[1] * w_my[1]

        rdma_c.wait()
        out_ref[...] = out_mine + crecv[...].astype(jnp.float32)

    return pl.pallas_call(
        body,
        out_shape=jax.ShapeDtypeStruct((t_loc, d), jnp.float32),
        in_specs=[
            pl.BlockSpec(memory_space=pltpu.VMEM),
            pl.BlockSpec(memory_space=pltpu.VMEM),
            pl.BlockSpec(memory_space=pl.ANY),
            pl.BlockSpec(memory_space=pl.ANY),
        ],
        out_specs=pl.BlockSpec(memory_space=pltpu.VMEM),
        scratch_shapes=[
            pltpu.VMEM(W1.shape, jnp.float32),
            pltpu.VMEM(W2.shape, jnp.float32),
            pltpu.VMEM(router.shape, jnp.float32),
            pltpu.VMEM((t_loc, d), jnp.bfloat16),
            pltpu.VMEM((t_loc, d), jnp.bfloat16),
            pltpu.VMEM((t_loc, d), jnp.bfloat16),
            pltpu.VMEM((t_loc, d), jnp.bfloat16),
            pltpu.SemaphoreType.DMA((2 * E_LOC,)),
            pltpu.SemaphoreType.DMA((3,)),
            pltpu.SemaphoreType.DMA((3,)),
        ],
        compiler_params=pltpu.CompilerParams(collective_id=0),
    )(x, router, W1, W2)


# device time: 22206 ns/iter; 1.3657x vs baseline; 1.0796x over previous
import jax
import jax.numpy as jnp
from jax import lax
from jax.experimental import pallas as pl
from jax.experimental.pallas import tpu as pltpu

E_LOC = 2


def kernel(x, router, W1, W2):
    t_loc, d = x.shape
    f = W1.shape[2]

    def body(x_ref, r_ref, w1_hbm, w2_hbm, out_ref,
             w1_vmem, w2_vmem, rr_buf, xsend, xrecv, csend, crecv,
             w_sems, send_sems, recv_sems):
        my_x = lax.axis_index("x")
        my_y = lax.axis_index("y")
        nbr = (1 - my_x, my_y)

        w_copies = []
        for e in range(E_LOC):
            c1 = pltpu.make_async_copy(w1_hbm.at[e], w1_vmem.at[e],
                                       w_sems.at[e])
            c1.start()
            c2 = pltpu.make_async_copy(w2_hbm.at[e], w2_vmem.at[e],
                                       w_sems.at[E_LOC + e])
            c2.start()
            w_copies.append((c1, c2))

        barrier_sem = pltpu.get_barrier_semaphore()
        pl.semaphore_signal(barrier_sem, inc=1, device_id=nbr,
                            device_id_type=pl.DeviceIdType.MESH)
        pl.semaphore_wait(barrier_sem, 1)

        xsend[...] = x_ref[...].astype(jnp.bfloat16)
        rdma_x = pltpu.make_async_remote_copy(
            src_ref=xsend, dst_ref=xrecv,
            send_sem=send_sems.at[0], recv_sem=recv_sems.at[0],
            device_id=nbr, device_id_type=pl.DeviceIdType.MESH)
        rdma_x.start()
        rdma_r = pltpu.make_async_remote_copy(
            src_ref=r_ref, dst_ref=rr_buf,
            send_sem=send_sems.at[1], recv_sem=recv_sems.at[1],
            device_id=nbr, device_id_type=pl.DeviceIdType.MESH)
        rdma_r.start()

        def topk_weights(g_loc, g_rem):
            g4 = jnp.concatenate([g_loc, g_rem], axis=1)
            m1 = jnp.max(g4, axis=1, keepdims=True)
            m2 = jnp.max(jnp.where(g4 == m1, -1e30, g4), axis=1,
                         keepdims=True)
            denom = 1.0 + jnp.exp(m2 - m1)
            return [jnp.where(g_loc[:, e:e + 1] >= m2,
                              jnp.exp(g_loc[:, e:e + 1] - m1) / denom, 0.0)
                    for e in range(E_LOC)]

        x_my = x_ref[...]
        w_copies[0][0].wait()
        h_my0 = jnp.maximum(
            jnp.dot(x_my, w1_vmem[0], preferred_element_type=jnp.float32),
            0.0)
        w_copies[1][0].wait()
        h_my1 = jnp.maximum(
            jnp.dot(x_my, w1_vmem[1], preferred_element_type=jnp.float32),
            0.0)
        w_copies[0][1].wait()
        c_my0 = jnp.dot(h_my0, w2_vmem[0],
                        preferred_element_type=jnp.float32)

        rdma_r.wait()
        rdma_x.wait()

        x_nb = xrecv[...].astype(jnp.float32)
        g_nb_loc = jnp.dot(x_nb, r_ref[...],
                           preferred_element_type=jnp.float32)
        g_nb_rem = jnp.dot(x_nb, rr_buf[...],
                           preferred_element_type=jnp.float32)
        w_nb = topk_weights(g_nb_loc, g_nb_rem)
        h_nb0 = jnp.maximum(
            jnp.dot(x_nb, w1_vmem[0], preferred_element_type=jnp.float32),
            0.0)
        c_nb0 = jnp.dot(h_nb0, w2_vmem[0],
                        preferred_element_type=jnp.float32)
        h_nb1 = jnp.maximum(
            jnp.dot(x_nb, w1_vmem[1], preferred_element_type=jnp.float32),
            0.0)
        w_copies[1][1].wait()
        c_nb1 = jnp.dot(h_nb1, w2_vmem[1],
                        preferred_element_type=jnp.float32)
        csend[...] = (c_nb0 * w_nb[0] + c_nb1 * w_nb[1]).astype(
            jnp.bfloat16)
        rdma_c = pltpu.make_async_remote_copy(
            src_ref=csend, dst_ref=crecv,
            send_sem=send_sems.at[2], recv_sem=recv_sems.at[2],
            device_id=nbr, device_id_type=pl.DeviceIdType.MESH)
        rdma_c.start()

        c_my1 = jnp.dot(h_my1, w2_vmem[1],
                        preferred_element_type=jnp.float32)
        g_my_loc = jnp.dot(x_my, r_ref[...],
                           preferred_element_type=jnp.float32)
        g_my_rem = jnp.dot(x_my, rr_buf[...],
                           preferred_element_type=jnp.float32)
        w_my = topk_weights(g_my_loc, g_my_rem)
        out_mine = c_my0 * w_my[0] + c_my1 * w_my[1]

        rdma_c.wait()
        out_ref[...] = out_mine + crecv[...].astype(jnp.float32)

    return pl.pallas_call(
        body,
        out_shape=jax.ShapeDtypeStruct((t_loc, d), jnp.float32),
        in_specs=[
            pl.BlockSpec(memory_space=pltpu.VMEM),
            pl.BlockSpec(memory_space=pltpu.VMEM),
            pl.BlockSpec(memory_space=pl.ANY),
            pl.BlockSpec(memory_space=pl.ANY),
        ],
        out_specs=pl.BlockSpec(memory_space=pltpu.VMEM),
        scratch_shapes=[
            pltpu.VMEM(W1.shape, jnp.float32),
            pltpu.VMEM(W2.shape, jnp.float32),
            pltpu.VMEM(router.shape, jnp.float32),
            pltpu.VMEM((t_loc, d), jnp.bfloat16),
            pltpu.VMEM((t_loc, d), jnp.bfloat16),
            pltpu.VMEM((t_loc, d), jnp.bfloat16),
            pltpu.VMEM((t_loc, d), jnp.bfloat16),
            pltpu.SemaphoreType.DMA((2 * E_LOC,)),
            pltpu.SemaphoreType.DMA((3,)),
            pltpu.SemaphoreType.DMA((3,)),
        ],
        compiler_params=pltpu.CompilerParams(collective_id=0),
    )(x, router, W1, W2)
